# baseline (device time: 184162 ns/iter reference)
import jax
import jax.numpy as jnp
from jax import lax
from jax.experimental import pallas as pl
from jax.experimental.pallas import tpu as pltpu

B, S, H, Dh, Dr = 4, 256, 32, 128, 64
M = B * S
D = 4096
DC = 128
BF = jnp.bfloat16
F32 = jnp.float32
SCALE = (Dh + Dr) ** -0.5

BN = 256
NQ = D // BN
NQR = (H * Dr) // BN


def _c_kr_body(x_ref, wdkv_ref, wkr_ref, c_ref, kr_ref, xb_ref):
    xv = x_ref[...].astype(BF)
    xb_ref[...] = xv
    c_ref[...] = jnp.dot(
        xv, wdkv_ref[...].astype(BF), preferred_element_type=F32
    ).astype(BF)
    kr_ref[...] = jnp.dot(
        xv, wkr_ref[...].astype(BF), preferred_element_type=F32
    ).astype(BF)


def _c_kr(x, wdkv, wkr):
    return pl.pallas_call(
        _c_kr_body,
        out_shape=[
            jax.ShapeDtypeStruct((M, DC), BF),
            jax.ShapeDtypeStruct((M, Dr), BF),
            jax.ShapeDtypeStruct((M, D), BF),
        ],
        in_specs=[
            pl.BlockSpec(memory_space=pltpu.VMEM),
            pl.BlockSpec(memory_space=pltpu.VMEM),
            pl.BlockSpec(memory_space=pltpu.VMEM),
        ],
        out_specs=[
            pl.BlockSpec(memory_space=pltpu.VMEM),
            pl.BlockSpec(memory_space=pltpu.VMEM),
            pl.BlockSpec(memory_space=pltpu.VMEM),
        ],
    )(x, wdkv, wkr)


def _fused_body(
    c_ref, wuk_ref, wuv_ref, xb_ref, wq_ref, wqr_ref,
    k_ref, v_ref, q_ref, qr_ref,
    c_full, wuk_full, wuv_full, send_sems, recv_sems,
):
    j = pl.program_id(0)
    nj = pl.num_programs(0)
    my_x = lax.axis_index("x")
    my_y = lax.axis_index("y")
    my_z = lax.axis_index("z")
    partner = (1 - my_x, my_y, my_z)

    def make_rdmas():
        rdmas = []
        for i, buf in enumerate((c_full, wuk_full, wuv_full)):
            rdmas.append(
                pltpu.make_async_remote_copy(
                    src_ref=buf.at[my_x],
                    dst_ref=buf.at[my_x],
                    send_sem=send_sems.at[i],
                    recv_sem=recv_sems.at[i],
                    device_id=partner,
                    device_id_type=pl.DeviceIdType.MESH,
                )
            )
        return rdmas

    @pl.when(j == 0)
    def _():
        c_full[my_x] = c_ref[...]
        wuk_full[my_x] = wuk_ref[...].astype(BF)
        wuv_full[my_x] = wuv_ref[...].astype(BF)
        barrier = pltpu.get_barrier_semaphore()
        pl.semaphore_signal(
            barrier, inc=1, device_id=partner,
            device_id_type=pl.DeviceIdType.MESH,
        )
        pl.semaphore_wait(barrier, 1)
        for rdma in make_rdmas():
            rdma.start()

    @pl.when(j < NQ)
    def _():
        q_ref[...] = (
            jnp.dot(xb_ref[...], wq_ref[...].astype(BF),
                    preferred_element_type=F32) * SCALE
        ).astype(BF)

    @pl.when(j >= NQ)
    def _():
        qr_ref[...] = (
            jnp.dot(xb_ref[...], wqr_ref[...].astype(BF),
                    preferred_element_type=F32) * SCALE
        ).astype(BF)

    @pl.when(j == nj - 1)
    def _():
        for rdma in make_rdmas():
            rdma.wait()
        ck = 1024
        for col in range(0, D, ck):
            cs = slice(col, col + ck)
            k_ref[:, cs] = (
                jnp.dot(c_full[0], wuk_full[0][:, cs],
                        preferred_element_type=F32)
                + jnp.dot(c_full[1], wuk_full[1][:, cs],
                          preferred_element_type=F32)
            ).astype(BF)
            v_ref[:, cs] = (
                jnp.dot(c_full[0], wuv_full[0][:, cs],
                        preferred_element_type=F32)
                + jnp.dot(c_full[1], wuv_full[1][:, cs],
                          preferred_element_type=F32)
            ).astype(BF)


def _fused_exchange(c, wuk, wuv, xb, wq, wqr):
    nqr = H * Dr
    return pl.pallas_call(
        _fused_body,
        grid=(NQ + NQR,),
        in_specs=[
            pl.BlockSpec((M, DC), lambda j: (0, 0)),
            pl.BlockSpec((DC, D), lambda j: (0, 0)),
            pl.BlockSpec((DC, D), lambda j: (0, 0)),
            pl.BlockSpec((M, D), lambda j: (0, 0)),
            pl.BlockSpec(
                (D, BN), lambda j: (0, jnp.minimum(j, NQ - 1))),
            pl.BlockSpec(
                (D, BN), lambda j: (0, jnp.maximum(j, NQ) - NQ)),
        ],
        out_specs=[
            pl.BlockSpec((M, D), lambda j: (0, 0)),
            pl.BlockSpec((M, D), lambda j: (0, 0)),
            pl.BlockSpec(
                (M, BN), lambda j: (0, jnp.minimum(j, NQ - 1))),
            pl.BlockSpec(
                (M, BN), lambda j: (0, jnp.maximum(j, NQ) - NQ)),
        ],
        out_shape=[
            jax.ShapeDtypeStruct((M, D), BF),
            jax.ShapeDtypeStruct((M, D), BF),
            jax.ShapeDtypeStruct((M, D), BF),
            jax.ShapeDtypeStruct((M, nqr), BF),
        ],
        scratch_shapes=[
            pltpu.VMEM((2, M, DC), BF),
            pltpu.VMEM((2, DC, D), BF),
            pltpu.VMEM((2, DC, D), BF),
            pltpu.SemaphoreType.DMA((3,)),
            pltpu.SemaphoreType.DMA((3,)),
        ],
        compiler_params=pltpu.CompilerParams(
            collective_id=0, vmem_limit_bytes=60 * 1024 * 1024
        ),
    )(c, wuk, wuv, xb, wq, wqr)


def _mm_body(a_ref, w_ref, o_ref):
    o_ref[...] = jnp.dot(
        a_ref[...], w_ref[...].astype(BF), preferred_element_type=F32
    )


def _mm_f32(a, w, block_n):
    m, k = a.shape
    _, n = w.shape
    return pl.pallas_call(
        _mm_body,
        grid=(n // block_n,),
        in_specs=[
            pl.BlockSpec((m, k), lambda j: (0, 0)),
            pl.BlockSpec((k, block_n), lambda j: (0, j)),
        ],
        out_specs=pl.BlockSpec((m, block_n), lambda j: (0, j)),
        out_shape=jax.ShapeDtypeStruct((m, n), F32),
    )(a, w)


def _attn_body(q_ref, k_ref, v_ref, qr_ref, kr_ref, o_ref):
    qr = qr_ref[...]
    kr = kr_ref[...]
    for h in range(H):
        qcat = jnp.concatenate(
            [q_ref[:, h * Dh:(h + 1) * Dh], qr[:, h * Dr:(h + 1) * Dr]],
            axis=1,
        )
        kcat = jnp.concatenate([k_ref[:, h * Dh:(h + 1) * Dh], kr], axis=1)
        s = lax.dot_general(
            qcat, kcat, (((1,), (1,)), ((), ())), preferred_element_type=F32
        )
        mx = jnp.max(s, axis=1, keepdims=True)
        p = jnp.exp(s - mx)
        p = p / jnp.sum(p, axis=1, keepdims=True)
        o_ref[:, h * Dh:(h + 1) * Dh] = jnp.dot(
            p.astype(BF), v_ref[:, h * Dh:(h + 1) * Dh],
            preferred_element_type=F32,
        ).astype(BF)


def _attention(q, k, v, qr, kr):
    return pl.pallas_call(
        _attn_body,
        grid=(B,),
        in_specs=[
            pl.BlockSpec((S, H * Dh), lambda b: (b, 0)),
            pl.BlockSpec((S, H * Dh), lambda b: (b, 0)),
            pl.BlockSpec((S, H * Dh), lambda b: (b, 0)),
            pl.BlockSpec((S, H * Dr), lambda b: (b, 0)),
            pl.BlockSpec((S, Dr), lambda b: (b, 0)),
        ],
        out_specs=pl.BlockSpec((S, H * Dh), lambda b: (b, 0)),
        out_shape=jax.ShapeDtypeStruct((M, H * Dh), BF),
    )(q, k, v, qr, kr)


def kernel(x, Wdkv, Wuk, Wuv, Wq, Wqr, Wkr, Wo):
    x2 = x.reshape(M, D)
    c, kr, xb = _c_kr(x2, Wdkv, Wkr)
    k, v, q, qr = _fused_exchange(c, Wuk, Wuv, xb, Wq, Wqr)
    o = _attention(q, k, v, qr, kr)
    out = _mm_f32(o, Wo, 512)
    return out.reshape(B, S, D)


# device time: 156796 ns/iter; 1.1745x vs baseline; 1.1745x over previous
import jax
import jax.numpy as jnp
from jax import lax
from jax.experimental import pallas as pl
from jax.experimental.pallas import tpu as pltpu

B, S, H, Dh, Dr = 4, 256, 32, 128, 64
M = B * S
D = 4096
DC = 128
BF = jnp.bfloat16
F32 = jnp.float32
SCALE = (Dh + Dr) ** -0.5

BN = 512
NQ = D // BN
NQR = (H * Dr) // BN
NWO = D // BN


def _c_kr_body(x_ref, wdkv_ref, wkr_ref, c_ref, kr_ref, xb_ref):
    xv = x_ref[...].astype(BF)
    xb_ref[...] = xv
    c_ref[...] = jnp.dot(
        xv, wdkv_ref[...].astype(BF), preferred_element_type=F32
    ).astype(BF)
    kr_ref[...] = jnp.dot(
        xv, wkr_ref[...].astype(BF), preferred_element_type=F32
    ).astype(BF)


def _c_kr(xg, wdkv, wkr):
    return pl.pallas_call(
        _c_kr_body,
        out_shape=[
            jax.ShapeDtypeStruct((S, DC), BF),
            jax.ShapeDtypeStruct((S, Dr), BF),
            jax.ShapeDtypeStruct((S, D), BF),
        ],
        in_specs=[pl.BlockSpec(memory_space=pltpu.VMEM)] * 3,
        out_specs=[pl.BlockSpec(memory_space=pltpu.VMEM)] * 3,
    )(xg, wdkv, wkr)


def _fused_body(
    c_ref, wuk_ref, wuv_ref, xb_ref, wq_ref, wqr_ref,
    k_ref, v_ref, q_ref, qr_ref,
    c_full, wuk_full, wuv_full, send_sems, recv_sems,
):
    j = pl.program_id(0)
    nj = pl.num_programs(0)
    my_x = lax.axis_index("x")
    my_y = lax.axis_index("y")
    my_z = lax.axis_index("z")
    partner = (1 - my_x, my_y, my_z)

    def make_rdmas():
        rdmas = []
        for i, buf in enumerate((c_full, wuk_full, wuv_full)):
            rdmas.append(
                pltpu.make_async_remote_copy(
                    src_ref=buf.at[my_x],
                    dst_ref=buf.at[my_x],
                    send_sem=send_sems.at[i],
                    recv_sem=recv_sems.at[i],
                    device_id=partner,
                    device_id_type=pl.DeviceIdType.MESH,
                )
            )
        return rdmas

    @pl.when(j == 0)
    def _():
        c_full[my_x] = c_ref[...]
        wuk_full[my_x] = wuk_ref[...].astype(BF)
        wuv_full[my_x] = wuv_ref[...].astype(BF)
        barrier = pltpu.get_barrier_semaphore()
        pl.semaphore_signal(
            barrier, inc=1, device_id=partner,
            device_id_type=pl.DeviceIdType.MESH,
        )
        pl.semaphore_wait(barrier, 1)
        for rdma in make_rdmas():
            rdma.start()

    @pl.when(j < NQ)
    def _():
        q_ref[...] = (
            jnp.dot(xb_ref[...], wq_ref[...].astype(BF),
                    preferred_element_type=F32) * SCALE
        ).astype(BF)

    @pl.when(j >= NQ)
    def _():
        qr_ref[...] = (
            jnp.dot(xb_ref[...], wqr_ref[...].astype(BF),
                    preferred_element_type=F32) * SCALE
        ).astype(BF)

    @pl.when(j == nj - 1)
    def _():
        for rdma in make_rdmas():
            rdma.wait()
        ck = 2048
        for col in range(0, D, ck):
            cs = slice(col, col + ck)
            k_ref[:, cs] = (
                jnp.dot(c_full[0], wuk_full[0][:, cs],
                        preferred_element_type=F32)
                + jnp.dot(c_full[1], wuk_full[1][:, cs],
                          preferred_element_type=F32)
            ).astype(BF)
            v_ref[:, cs] = (
                jnp.dot(c_full[0], wuv_full[0][:, cs],
                        preferred_element_type=F32)
                + jnp.dot(c_full[1], wuv_full[1][:, cs],
                          preferred_element_type=F32)
            ).astype(BF)


def _fused_exchange(c, wuk, wuv, xb, wq, wqr):
    nqr = H * Dr
    return pl.pallas_call(
        _fused_body,
        grid=(NQ + NQR,),
        in_specs=[
            pl.BlockSpec((S, DC), lambda j: (0, 0)),
            pl.BlockSpec((DC, D), lambda j: (0, 0)),
            pl.BlockSpec((DC, D), lambda j: (0, 0)),
            pl.BlockSpec((S, D), lambda j: (0, 0)),
            pl.BlockSpec(
                (D, BN), lambda j: (0, jnp.minimum(j, NQ - 1))),
            pl.BlockSpec(
                (D, BN), lambda j: (0, jnp.maximum(j, NQ) - NQ)),
        ],
        out_specs=[
            pl.BlockSpec((S, D), lambda j: (0, 0)),
            pl.BlockSpec((S, D), lambda j: (0, 0)),
            pl.BlockSpec(
                (S, BN), lambda j: (0, jnp.minimum(j, NQ - 1))),
            pl.BlockSpec(
                (S, BN), lambda j: (0, jnp.maximum(j, NQ) - NQ)),
        ],
        out_shape=[
            jax.ShapeDtypeStruct((S, D), BF),
            jax.ShapeDtypeStruct((S, D), BF),
            jax.ShapeDtypeStruct((S, D), BF),
            jax.ShapeDtypeStruct((S, nqr), BF),
        ],
        scratch_shapes=[
            pltpu.VMEM((2, S, DC), BF),
            pltpu.VMEM((2, DC, D), BF),
            pltpu.VMEM((2, DC, D), BF),
            pltpu.SemaphoreType.DMA((3,)),
            pltpu.SemaphoreType.DMA((3,)),
        ],
        compiler_params=pltpu.CompilerParams(
            collective_id=0, vmem_limit_bytes=60 * 1024 * 1024
        ),
    )(c, wuk, wuv, xb, wq, wqr)


def _attn_wo_body(
    q_ref, k_ref, v_ref, qr_ref, kr_ref, wo_ref,
    out_ref,
    o_scr, gath, send_sems, recv_sems,
):
    j = pl.program_id(0)
    nj = pl.num_programs(0)
    my_x = lax.axis_index("x")
    my_y = lax.axis_index("y")
    my_z = lax.axis_index("z")
    my_g = 2 * my_y + my_z
    peers = [
        ((my_x, 1 - my_y, my_z), 2 * (1 - my_y) + my_z),
        ((my_x, my_y, 1 - my_z), 2 * my_y + (1 - my_z)),
        ((my_x, 1 - my_y, 1 - my_z), 2 * (1 - my_y) + (1 - my_z)),
    ]

    def make_rdmas():
        rdmas = []
        for i, (dev, g_p) in enumerate(peers):
            rdmas.append(
                pltpu.make_async_remote_copy(
                    src_ref=gath.at[my_g],
                    dst_ref=gath.at[my_g],
                    send_sem=send_sems.at[i],
                    recv_sem=recv_sems.at[i],
                    device_id=dev,
                    device_id_type=pl.DeviceIdType.MESH,
                )
            )
        return rdmas

    @pl.when(j == 0)
    def _():
        barrier = pltpu.get_barrier_semaphore()
        for dev, _ in peers:
            pl.semaphore_signal(
                barrier, inc=1, device_id=dev,
                device_id_type=pl.DeviceIdType.MESH,
            )
        pl.semaphore_wait(barrier, 3)
        qr = qr_ref[...]
        kr = kr_ref[...]
        for h in range(H):
            qcat = jnp.concatenate(
                [q_ref[:, h * Dh:(h + 1) * Dh],
                 qr[:, h * Dr:(h + 1) * Dr]], axis=1)
            kcat = jnp.concatenate(
                [k_ref[:, h * Dh:(h + 1) * Dh], kr], axis=1)
            s = lax.dot_general(
                qcat, kcat, (((1,), (1,)), ((), ())),
                preferred_element_type=F32,
            )
            mx = jnp.max(s, axis=1, keepdims=True)
            p = jnp.exp(s - mx)
            p = p / jnp.sum(p, axis=1, keepdims=True)
            o_scr[:, h * Dh:(h + 1) * Dh] = jnp.dot(
                p.astype(BF), v_ref[:, h * Dh:(h + 1) * Dh],
                preferred_element_type=F32,
            ).astype(BF)

    gath[my_g, :, pl.ds(j * BN, BN)] = jnp.dot(
        o_scr[...], wo_ref[...].astype(BF), preferred_element_type=F32
    ).astype(BF)

    @pl.when(j == nj - 1)
    def _():
        rdmas = make_rdmas()
        for rdma in rdmas:
            rdma.start()
        for rdma in rdmas:
            rdma.wait()
        for b in range(B):
            out_ref[b * S:(b + 1) * S, :] = gath[b].astype(F32)


def _attn_wo_gather(q, k, v, qr, kr, wo):
    return pl.pallas_call(
        _attn_wo_body,
        grid=(NWO,),
        in_specs=[
            pl.BlockSpec((S, H * Dh), lambda j: (0, 0)),
            pl.BlockSpec((S, H * Dh), lambda j: (0, 0)),
            pl.BlockSpec((S, H * Dh), lambda j: (0, 0)),
            pl.BlockSpec((S, H * Dr), lambda j: (0, 0)),
            pl.BlockSpec((S, Dr), lambda j: (0, 0)),
            pl.BlockSpec((D, BN), lambda j: (0, j)),
        ],
        out_specs=pl.BlockSpec((M, D), lambda j: (0, 0)),
        out_shape=jax.ShapeDtypeStruct((M, D), F32),
        scratch_shapes=[
            pltpu.VMEM((S, D), BF),
            pltpu.VMEM((B, S, D), BF),
            pltpu.SemaphoreType.DMA((3,)),
            pltpu.SemaphoreType.DMA((3,)),
        ],
        compiler_params=pltpu.CompilerParams(
            collective_id=1, vmem_limit_bytes=60 * 1024 * 1024
        ),
    )(q, k, v, qr, kr, wo)


def kernel(x, Wdkv, Wuk, Wuv, Wq, Wqr, Wkr, Wo):
    my_y = lax.axis_index("y")
    my_z = lax.axis_index("z")
    g = 2 * my_y + my_z
    xg = lax.dynamic_index_in_dim(x, g, axis=0, keepdims=False)
    c, kr, xb = _c_kr(xg, Wdkv, Wkr)
    k, v, q, qr = _fused_exchange(c, Wuk, Wuv, xb, Wq, Wqr)
    out = _attn_wo_gather(q, k, v, qr, kr, Wo)
    return out.reshape(B, S, D)


# device time: 139637 ns/iter; 1.3189x vs baseline; 1.1229x over previous
import jax
import jax.numpy as jnp
from jax import lax
from jax.experimental import pallas as pl
from jax.experimental.pallas import tpu as pltpu

B, S, H, Dh, Dr = 4, 256, 32, 128, 64
M = B * S
D = 4096
DC = 128
BF = jnp.bfloat16
F32 = jnp.float32
SCALE = (Dh + Dr) ** -0.5

BN = 512
NQ = D // BN
NQR = (H * Dr) // BN
NWO = D // BN


def _c_kr_body(x_ref, wdkv_ref, wkr_ref, c_ref, kr_ref, xb_ref):
    xv = x_ref[...].astype(BF)
    xb_ref[...] = xv
    c_ref[...] = jnp.dot(
        xv, wdkv_ref[...].astype(BF), preferred_element_type=F32
    ).astype(BF)
    kr_ref[...] = jnp.dot(
        xv, wkr_ref[...].astype(BF), preferred_element_type=F32
    ).astype(BF)


def _c_kr(xg, wdkv, wkr):
    return pl.pallas_call(
        _c_kr_body,
        out_shape=[
            jax.ShapeDtypeStruct((S, DC), BF),
            jax.ShapeDtypeStruct((S, Dr), BF),
            jax.ShapeDtypeStruct((S, D), BF),
        ],
        in_specs=[pl.BlockSpec(memory_space=pltpu.VMEM)] * 3,
        out_specs=[pl.BlockSpec(memory_space=pltpu.VMEM)] * 3,
    )(xg, wdkv, wkr)


def _fused_body(
    c_ref, wuk_ref, wuv_ref, xb_ref, wq_ref, wqr_ref,
    k_ref, v_ref, q_ref, qr_ref,
    c_full, wuk_full, wuv_full, send_sems, recv_sems,
):
    j = pl.program_id(0)
    nj = pl.num_programs(0)
    my_x = lax.axis_index("x")
    my_y = lax.axis_index("y")
    my_z = lax.axis_index("z")
    partner = (1 - my_x, my_y, my_z)

    def make_rdmas():
        rdmas = []
        for i, buf in enumerate((c_full, wuk_full, wuv_full)):
            rdmas.append(
                pltpu.make_async_remote_copy(
                    src_ref=buf.at[my_x],
                    dst_ref=buf.at[my_x],
                    send_sem=send_sems.at[i],
                    recv_sem=recv_sems.at[i],
                    device_id=partner,
                    device_id_type=pl.DeviceIdType.MESH,
                )
            )
        return rdmas

    @pl.when(j == 0)
    def _():
        c_full[my_x] = c_ref[...]
        wuk_full[my_x] = wuk_ref[...].astype(BF)
        wuv_full[my_x] = wuv_ref[...].astype(BF)
        barrier = pltpu.get_barrier_semaphore()
        pl.semaphore_signal(
            barrier, inc=1, device_id=partner,
            device_id_type=pl.DeviceIdType.MESH,
        )
        pl.semaphore_wait(barrier, 1)
        for rdma in make_rdmas():
            rdma.start()

    @pl.when(j < NQ)
    def _():
        q_ref[...] = (
            jnp.dot(xb_ref[...], wq_ref[...].astype(BF),
                    preferred_element_type=F32) * SCALE
        ).astype(BF)

    @pl.when(j >= NQ)
    def _():
        qr_ref[...] = (
            jnp.dot(xb_ref[...], wqr_ref[...].astype(BF),
                    preferred_element_type=F32) * SCALE
        ).astype(BF)

    @pl.when(j == nj - 1)
    def _():
        for rdma in make_rdmas():
            rdma.wait()
        ck = 2048
        for col in range(0, D, ck):
            cs = slice(col, col + ck)
            k_ref[:, cs] = (
                jnp.dot(c_full[0], wuk_full[0][:, cs],
                        preferred_element_type=F32)
                + jnp.dot(c_full[1], wuk_full[1][:, cs],
                          preferred_element_type=F32)
            ).astype(BF)
            v_ref[:, cs] = (
                jnp.dot(c_full[0], wuv_full[0][:, cs],
                        preferred_element_type=F32)
                + jnp.dot(c_full[1], wuv_full[1][:, cs],
                          preferred_element_type=F32)
            ).astype(BF)


def _fused_exchange(c, wuk, wuv, xb, wq, wqr):
    nqr = H * Dr
    return pl.pallas_call(
        _fused_body,
        grid=(NQ + NQR,),
        in_specs=[
            pl.BlockSpec((S, DC), lambda j: (0, 0)),
            pl.BlockSpec((DC, D), lambda j: (0, 0)),
            pl.BlockSpec((DC, D), lambda j: (0, 0)),
            pl.BlockSpec((S, D), lambda j: (0, 0)),
            pl.BlockSpec(
                (D, BN), lambda j: (0, jnp.minimum(j, NQ - 1))),
            pl.BlockSpec(
                (D, BN), lambda j: (0, jnp.maximum(j, NQ) - NQ)),
        ],
        out_specs=[
            pl.BlockSpec((S, D), lambda j: (0, 0)),
            pl.BlockSpec((S, D), lambda j: (0, 0)),
            pl.BlockSpec(
                (S, BN), lambda j: (0, jnp.minimum(j, NQ - 1))),
            pl.BlockSpec(
                (S, BN), lambda j: (0, jnp.maximum(j, NQ) - NQ)),
        ],
        out_shape=[
            jax.ShapeDtypeStruct((S, D), BF),
            jax.ShapeDtypeStruct((S, D), BF),
            jax.ShapeDtypeStruct((S, D), BF),
            jax.ShapeDtypeStruct((S, nqr), BF),
        ],
        scratch_shapes=[
            pltpu.VMEM((2, S, DC), BF),
            pltpu.VMEM((2, DC, D), BF),
            pltpu.VMEM((2, DC, D), BF),
            pltpu.SemaphoreType.DMA((3,)),
            pltpu.SemaphoreType.DMA((3,)),
        ],
        compiler_params=pltpu.CompilerParams(
            collective_id=0, vmem_limit_bytes=60 * 1024 * 1024
        ),
    )(c, wuk, wuv, xb, wq, wqr)


def _attn_wo_body(
    q_ref, k_ref, v_ref, qr_ref, kr_ref, wo_ref,
    out_ref,
    o_scr, gath, send_sems, recv_sems,
):
    j = pl.program_id(0)
    nj = pl.num_programs(0)
    my_x = lax.axis_index("x")
    my_y = lax.axis_index("y")
    my_z = lax.axis_index("z")
    my_g = 2 * my_y + my_z
    peers = [
        ((my_x, 1 - my_y, my_z), 2 * (1 - my_y) + my_z),
        ((my_x, my_y, 1 - my_z), 2 * my_y + (1 - my_z)),
        ((my_x, 1 - my_y, 1 - my_z), 2 * (1 - my_y) + (1 - my_z)),
    ]

    def chunk_rdmas(jj, slot):
        rdmas = []
        for i, (dev, g_p) in enumerate(peers):
            rdmas.append(
                pltpu.make_async_remote_copy(
                    src_ref=gath.at[my_g, :, pl.ds(jj * BN, BN)],
                    dst_ref=gath.at[my_g, :, pl.ds(jj * BN, BN)],
                    send_sem=send_sems.at[i, slot],
                    recv_sem=recv_sems.at[i, slot],
                    device_id=dev,
                    device_id_type=pl.DeviceIdType.MESH,
                )
            )
        return rdmas

    @pl.when(j == 0)
    def _():
        barrier = pltpu.get_barrier_semaphore()
        for dev, _ in peers:
            pl.semaphore_signal(
                barrier, inc=1, device_id=dev,
                device_id_type=pl.DeviceIdType.MESH,
            )
        pl.semaphore_wait(barrier, 3)
        qr = qr_ref[...]
        kr = kr_ref[...]
        for h in range(H):
            qcat = jnp.concatenate(
                [q_ref[:, h * Dh:(h + 1) * Dh],
                 qr[:, h * Dr:(h + 1) * Dr]], axis=1)
            kcat = jnp.concatenate(
                [k_ref[:, h * Dh:(h + 1) * Dh], kr], axis=1)
            s = lax.dot_general(
                qcat, kcat, (((1,), (1,)), ((), ())),
                preferred_element_type=F32,
            )
            mx = jnp.max(s, axis=1, keepdims=True)
            p = jnp.exp(s - mx)
            p = p / jnp.sum(p, axis=1, keepdims=True)
            o_scr[:, h * Dh:(h + 1) * Dh] = jnp.dot(
                p.astype(BF), v_ref[:, h * Dh:(h + 1) * Dh],
                preferred_element_type=F32,
            ).astype(BF)

    gath[my_g, :, pl.ds(j * BN, BN)] = jnp.dot(
        o_scr[...], wo_ref[...].astype(BF), preferred_element_type=F32
    ).astype(BF)
    for rdma in chunk_rdmas(j, j):
        rdma.start()

    @pl.when(j == nj - 1)
    def _():
        for slot in range(NWO):
            for rdma in chunk_rdmas(slot, slot):
                rdma.wait()
        for b in range(B):
            out_ref[b * S:(b + 1) * S, :] = gath[b].astype(F32)


def _attn_wo_gather(q, k, v, qr, kr, wo):
    return pl.pallas_call(
        _attn_wo_body,
        grid=(NWO,),
        in_specs=[
            pl.BlockSpec((S, H * Dh), lambda j: (0, 0)),
            pl.BlockSpec((S, H * Dh), lambda j: (0, 0)),
            pl.BlockSpec((S, H * Dh), lambda j: (0, 0)),
            pl.BlockSpec((S, H * Dr), lambda j: (0, 0)),
            pl.BlockSpec((S, Dr), lambda j: (0, 0)),
            pl.BlockSpec((D, BN), lambda j: (0, j)),
        ],
        out_specs=pl.BlockSpec((M, D), lambda j: (0, 0)),
        out_shape=jax.ShapeDtypeStruct((M, D), F32),
        scratch_shapes=[
            pltpu.VMEM((S, D), BF),
            pltpu.VMEM((B, S, D), BF),
            pltpu.SemaphoreType.DMA((3, NWO)),
            pltpu.SemaphoreType.DMA((3, NWO)),
        ],
        compiler_params=pltpu.CompilerParams(
            collective_id=1, vmem_limit_bytes=60 * 1024 * 1024
        ),
    )(q, k, v, qr, kr, wo)


def kernel(x, Wdkv, Wuk, Wuv, Wq, Wqr, Wkr, Wo):
    my_y = lax.axis_index("y")
    my_z = lax.axis_index("z")
    g = 2 * my_y + my_z
    xg = lax.dynamic_index_in_dim(x, g, axis=0, keepdims=False)
    c, kr, xb = _c_kr(xg, Wdkv, Wkr)
    k, v, q, qr = _fused_exchange(c, Wuk, Wuv, xb, Wq, Wqr)
    out = _attn_wo_gather(q, k, v, qr, kr, Wo)
    return out.reshape(B, S, D)


# device time: 139177 ns/iter; 1.3232x vs baseline; 1.0033x over previous
import jax
import jax.numpy as jnp
from jax import lax
from jax.experimental import pallas as pl
from jax.experimental.pallas import tpu as pltpu

B, S, H, Dh, Dr = 4, 256, 32, 128, 64
M = B * S
D = 4096
DC = 128
BF = jnp.bfloat16
F32 = jnp.float32
SCALE = (Dh + Dr) ** -0.5

BN = 512
NQ = D // BN
NQR = (H * Dr) // BN
NWO = D // BN


def _c_kr_body(x_ref, wdkv_ref, wkr_ref, c_ref, kr_ref, xb_ref):
    xv = x_ref[...].astype(BF)
    xb_ref[...] = xv
    c_ref[...] = jnp.dot(
        xv, wdkv_ref[...].astype(BF), preferred_element_type=F32
    ).astype(BF)
    kr_ref[...] = jnp.dot(
        xv, wkr_ref[...].astype(BF), preferred_element_type=F32
    ).astype(BF)


def _c_kr(xg, wdkv, wkr):
    return pl.pallas_call(
        _c_kr_body,
        out_shape=[
            jax.ShapeDtypeStruct((S, DC), BF),
            jax.ShapeDtypeStruct((S, Dr), BF),
            jax.ShapeDtypeStruct((S, D), BF),
        ],
        in_specs=[pl.BlockSpec(memory_space=pltpu.VMEM)] * 3,
        out_specs=[pl.BlockSpec(memory_space=pltpu.VMEM)] * 3,
    )(xg, wdkv, wkr)


def _fused_body(
    c_ref, wuk_ref, wuv_ref, xb_ref, wq_ref, wqr_ref,
    k_ref, v_ref, q_ref, qr_ref,
    c_full, wuk_full, wuv_full, send_sems, recv_sems,
):
    j = pl.program_id(0)
    nj = pl.num_programs(0)
    my_x = lax.axis_index("x")
    my_y = lax.axis_index("y")
    my_z = lax.axis_index("z")
    partner = (1 - my_x, my_y, my_z)

    def make_rdmas():
        rdmas = []
        for i, buf in enumerate((c_full, wuk_full, wuv_full)):
            rdmas.append(
                pltpu.make_async_remote_copy(
                    src_ref=buf.at[my_x],
                    dst_ref=buf.at[my_x],
                    send_sem=send_sems.at[i],
                    recv_sem=recv_sems.at[i],
                    device_id=partner,
                    device_id_type=pl.DeviceIdType.MESH,
                )
            )
        return rdmas

    @pl.when(j == 0)
    def _():
        c_full[my_x] = c_ref[...]
        wuk_full[my_x] = wuk_ref[...].astype(BF)
        wuv_full[my_x] = wuv_ref[...].astype(BF)
        barrier = pltpu.get_barrier_semaphore()
        pl.semaphore_signal(
            barrier, inc=1, device_id=partner,
            device_id_type=pl.DeviceIdType.MESH,
        )
        pl.semaphore_wait(barrier, 1)
        for rdma in make_rdmas():
            rdma.start()

    @pl.when(j < NQ)
    def _():
        q_ref[...] = (
            jnp.dot(xb_ref[...], wq_ref[...].astype(BF),
                    preferred_element_type=F32) * SCALE
        ).astype(BF)

    @pl.when(j >= NQ)
    def _():
        qr_ref[...] = (
            jnp.dot(xb_ref[...], wqr_ref[...].astype(BF),
                    preferred_element_type=F32) * SCALE
        ).astype(BF)

    @pl.when(j == nj - 1)
    def _():
        for rdma in make_rdmas():
            rdma.wait()
        ck = 2048
        for col in range(0, D, ck):
            cs = slice(col, col + ck)
            k_ref[:, cs] = (
                jnp.dot(c_full[0], wuk_full[0][:, cs],
                        preferred_element_type=F32)
                + jnp.dot(c_full[1], wuk_full[1][:, cs],
                          preferred_element_type=F32)
            ).astype(BF)
            v_ref[:, cs] = (
                jnp.dot(c_full[0], wuv_full[0][:, cs],
                        preferred_element_type=F32)
                + jnp.dot(c_full[1], wuv_full[1][:, cs],
                          preferred_element_type=F32)
            ).astype(BF)


def _fused_exchange(c, wuk, wuv, xb, wq, wqr):
    nqr = H * Dr
    return pl.pallas_call(
        _fused_body,
        grid=(NQ + NQR,),
        in_specs=[
            pl.BlockSpec((S, DC), lambda j: (0, 0)),
            pl.BlockSpec((DC, D), lambda j: (0, 0)),
            pl.BlockSpec((DC, D), lambda j: (0, 0)),
            pl.BlockSpec((S, D), lambda j: (0, 0)),
            pl.BlockSpec(
                (D, BN), lambda j: (0, jnp.minimum(j, NQ - 1))),
            pl.BlockSpec(
                (D, BN), lambda j: (0, jnp.maximum(j, NQ) - NQ)),
        ],
        out_specs=[
            pl.BlockSpec((S, D), lambda j: (0, 0)),
            pl.BlockSpec((S, D), lambda j: (0, 0)),
            pl.BlockSpec(
                (S, BN), lambda j: (0, jnp.minimum(j, NQ - 1))),
            pl.BlockSpec(
                (S, BN), lambda j: (0, jnp.maximum(j, NQ) - NQ)),
        ],
        out_shape=[
            jax.ShapeDtypeStruct((S, D), BF),
            jax.ShapeDtypeStruct((S, D), BF),
            jax.ShapeDtypeStruct((S, D), BF),
            jax.ShapeDtypeStruct((S, nqr), BF),
        ],
        scratch_shapes=[
            pltpu.VMEM((2, S, DC), BF),
            pltpu.VMEM((2, DC, D), BF),
            pltpu.VMEM((2, DC, D), BF),
            pltpu.SemaphoreType.DMA((3,)),
            pltpu.SemaphoreType.DMA((3,)),
        ],
        compiler_params=pltpu.CompilerParams(
            collective_id=0, vmem_limit_bytes=60 * 1024 * 1024
        ),
    )(c, wuk, wuv, xb, wq, wqr)


def _attn_wo_body(
    q_ref, k_ref, v_ref, qr_ref, kr_ref, wo_ref,
    out_ref,
    o_scr, gath, send_sems, recv_sems,
):
    j = pl.program_id(0)
    nj = pl.num_programs(0)
    my_x = lax.axis_index("x")
    my_y = lax.axis_index("y")
    my_z = lax.axis_index("z")
    my_g = 2 * my_y + my_z
    peers = [
        ((my_x, 1 - my_y, my_z), 2 * (1 - my_y) + my_z),
        ((my_x, my_y, 1 - my_z), 2 * my_y + (1 - my_z)),
        ((my_x, 1 - my_y, 1 - my_z), 2 * (1 - my_y) + (1 - my_z)),
    ]

    def chunk_rdmas(jj, slot):
        rdmas = []
        for i, (dev, g_p) in enumerate(peers):
            rdmas.append(
                pltpu.make_async_remote_copy(
                    src_ref=gath.at[my_g, :, pl.ds(jj * BN, BN)],
                    dst_ref=gath.at[my_g, :, pl.ds(jj * BN, BN)],
                    send_sem=send_sems.at[i, slot],
                    recv_sem=recv_sems.at[i, slot],
                    device_id=dev,
                    device_id_type=pl.DeviceIdType.MESH,
                )
            )
        return rdmas

    @pl.when(j == 0)
    def _():
        barrier = pltpu.get_barrier_semaphore()
        for dev, _ in peers:
            pl.semaphore_signal(
                barrier, inc=1, device_id=dev,
                device_id_type=pl.DeviceIdType.MESH,
            )
        pl.semaphore_wait(barrier, 3)
        qr = qr_ref[...]
        kr = kr_ref[...]
        for h in range(H):
            qcat = jnp.concatenate(
                [q_ref[:, h * Dh:(h + 1) * Dh],
                 qr[:, h * Dr:(h + 1) * Dr]], axis=1)
            kcat = jnp.concatenate(
                [k_ref[:, h * Dh:(h + 1) * Dh], kr], axis=1)
            s = lax.dot_general(
                qcat, kcat, (((1,), (1,)), ((), ())),
                preferred_element_type=F32,
            )
            mx = jnp.max(s, axis=1, keepdims=True)
            p = jnp.exp(s - mx)
            p = p / jnp.sum(p, axis=1, keepdims=True)
            o_scr[:, h * Dh:(h + 1) * Dh] = jnp.dot(
                p.astype(BF), v_ref[:, h * Dh:(h + 1) * Dh],
                preferred_element_type=F32,
            ).astype(BF)

    gath[my_g, :, pl.ds(j * BN, BN)] = jnp.dot(
        o_scr[...], wo_ref[...].astype(BF), preferred_element_type=F32
    ).astype(BF)
    for rdma in chunk_rdmas(j, j):
        rdma.start()

    @pl.when(j == nj - 1)
    def _():
        out_ref[my_g] = gath[my_g].astype(F32)
        for slot in range(NWO):
            rdmas = chunk_rdmas(slot, slot)
            for i in range(len(peers)):
                rdmas[i].wait_send()
        for i, (dev, g_p) in enumerate(peers):
            for slot in range(NWO):
                chunk_rdmas(slot, slot)[i].wait_recv()
            out_ref[g_p] = gath[g_p].astype(F32)


def _attn_wo_gather(q, k, v, qr, kr, wo):
    return pl.pallas_call(
        _attn_wo_body,
        grid=(NWO,),
        in_specs=[
            pl.BlockSpec((S, H * Dh), lambda j: (0, 0)),
            pl.BlockSpec((S, H * Dh), lambda j: (0, 0)),
            pl.BlockSpec((S, H * Dh), lambda j: (0, 0)),
            pl.BlockSpec((S, H * Dr), lambda j: (0, 0)),
            pl.BlockSpec((S, Dr), lambda j: (0, 0)),
            pl.BlockSpec((D, BN), lambda j: (0, j)),
        ],
        out_specs=pl.BlockSpec((B, S, D), lambda j: (0, 0, 0)),
        out_shape=jax.ShapeDtypeStruct((B, S, D), F32),
        scratch_shapes=[
            pltpu.VMEM((S, D), BF),
            pltpu.VMEM((B, S, D), BF),
            pltpu.SemaphoreType.DMA((3, NWO)),
            pltpu.SemaphoreType.DMA((3, NWO)),
        ],
        compiler_params=pltpu.CompilerParams(
            collective_id=1, vmem_limit_bytes=60 * 1024 * 1024
        ),
    )(q, k, v, qr, kr, wo)


def kernel(x, Wdkv, Wuk, Wuv, Wq, Wqr, Wkr, Wo):
    my_y = lax.axis_index("y")
    my_z = lax.axis_index("z")
    g = 2 * my_y + my_z
    xg = lax.dynamic_index_in_dim(x, g, axis=0, keepdims=False)
    c, kr, xb = _c_kr(xg, Wdkv, Wkr)
    k, v, q, qr = _fused_exchange(c, Wuk, Wuv, xb, Wq, Wqr)
    return _attn_wo_gather(q, k, v, qr, kr, Wo)
